# baseline (speedup 1.0000x reference)
import functools

import jax
import jax.numpy as jnp
from jax import lax
from jax.experimental import pallas as pl
from jax.experimental.pallas import tpu as pltpu
from jax.experimental.pallas import tpu_sc as plsc

V, D, B, K = 1000000, 64, 16384, 20
DP = 128
PACK = 4
SEG = 1 << 18
SEG_SHIFT = 18
WPR = DP // PACK
TW = 2048
SEGB = SEG // TW
NC, NS = 2, 16
NW = NC * NS
BPW = B // NW
C = 16
NCHUNK = BPW // C
G = 64
NEG_PER_CHUNK = C * K
NG = NEG_PER_CHUNK // G
L = 16
QB = 8


def _detranspose_body(*refs):
    ins, outs = refs[:2 * PACK], refs[2 * PACK:]
    for t in range(2):
        zs = []
        for p in range(PACK):
            xi = lax.bitcast_convert_type(ins[t * PACK + p][...], jnp.uint32)
            zs.append((xi[0:WPR, :] >> 16)
                      | (xi[WPR:2 * WPR, :] & jnp.uint32(0xFFFF0000)))
        zc = jnp.concatenate(zs, axis=0)
        outs[t][...] = lax.bitcast_convert_type(zc.T, jnp.float32)


def _detranspose(u_table, v_table):
    max_blk = V // TW
    in_specs = []
    for _ in range(2):
        for p in range(PACK):
            in_specs.append(pl.BlockSpec(
                (D, TW),
                functools.partial(
                    lambda p_, j: (0, jnp.minimum(j + SEGB * p_, max_blk)), p)))
    return pl.pallas_call(
        _detranspose_body,
        grid=(SEGB,),
        compiler_params=pltpu.CompilerParams(
            fuse_transposed_lhs_in_matmul=True),
        in_specs=in_specs,
        out_specs=[
            pl.BlockSpec((TW, DP), lambda j: (j, 0)),
            pl.BlockSpec((TW, DP), lambda j: (j, 0)),
        ],
        out_shape=(
            jax.ShapeDtypeStruct((SEG, DP), jnp.float32),
            jax.ShapeDtypeStruct((SEG, DP), jnp.float32),
        ),
    )(*([u_table.T] * PACK + [v_table.T] * PACK))


def _unpack2(vec):
    return plsc.unpack(plsc.bitcast(vec, jnp.bfloat16),
                       format=plsc.PackFormat.INTERLEAVED)


def _sc_dots(pur_hbm, puo_hbm, pvr_hbm, pvo_hbm, ngr_hbm, ngo_hbm,
             ut_hbm, vt_hbm,
             pd_hbm, nd_hbm,
             pur, puo, pvr, pvo, ngr, ngo,
             u_rows, v_rows, n_rows, acc_scr, pos_scr, pd, nd_c,
             sem0, sem1, wsem):
    w = lax.axis_index("s") * NC + lax.axis_index("c")
    pltpu.sync_copy(pur_hbm.at[w], pur)
    pltpu.sync_copy(puo_hbm.at[w], puo)
    pltpu.sync_copy(pvr_hbm.at[w], pvr)
    pltpu.sync_copy(pvo_hbm.at[w], pvo)
    pltpu.sync_copy(ngr_hbm.at[w], ngr)
    pltpu.sync_copy(ngo_hbm.at[w], ngo)

    sems = (sem0, sem1)
    lanes = lax.iota(jnp.int32, L)

    def issue(c, buf, sem):
        pltpu.async_copy(ut_hbm.at[pur.at[c]], u_rows.at[buf], sem)
        pltpu.async_copy(vt_hbm.at[pvr.at[c]], v_rows.at[buf], sem)
        for g in range(NG):
            pltpu.async_copy(
                vt_hbm.at[ngr.at[c * NG + g]],
                n_rows.at[buf].at[pl.ds(g * G, G)], sem)

    def drain(buf, sem):
        pltpu.make_async_copy(ut_hbm.at[pl.ds(0, C)], u_rows.at[buf], sem).wait()
        pltpu.make_async_copy(ut_hbm.at[pl.ds(0, C)], v_rows.at[buf], sem).wait()
        pltpu.make_async_copy(vt_hbm.at[pl.ds(0, NEG_PER_CHUNK)],
                              n_rows.at[buf], sem).wait()

    def tree_reduce(scr, base):
        gs = [plsc.load_gather(scr, [lanes * L + (base + j)]) for j in range(L)]
        while len(gs) > 1:
            gs = [gs[i] + gs[i + 1] for i in range(0, len(gs), 2)]
        return gs[0]

    def compute(c, buf):
        def q_body(q, _):
            pov = puo[c, pl.ds(q * QB, L)]
            pvv = pvo[c, pl.ds(q * QB, L)]
            for bl in range(QB):
                b = q * QB + bl
                ob = pov[bl]
                ua = plsc.bitcast(u_rows[buf, b, pl.ds(ob, L)], jnp.bfloat16)
                ub = plsc.bitcast(u_rows[buf, b, pl.ds(ob + L, L)], jnp.bfloat16)
                ov = pvv[bl]
                va = plsc.bitcast(v_rows[buf, b, pl.ds(ov, L)], jnp.bfloat16)
                vb = plsc.bitcast(v_rows[buf, b, pl.ds(ov + L, L)], jnp.bfloat16)
                p0, p1 = plsc.unpack(ua * va + ub * vb,
                                     format=plsc.PackFormat.INTERLEAVED)
                pos_scr[pl.ds(b * L, L)] = p0 + p1
                r0 = b * K
                nov1 = ngo[c, pl.ds(r0, L)]
                nov2 = ngo[c, pl.ds(r0 + L, L)]
                for k in range(K):
                    on = nov1[k] if k < L else nov2[k - L]
                    na = plsc.bitcast(
                        n_rows[buf, r0 + k, pl.ds(on, L)], jnp.bfloat16)
                    nb = plsc.bitcast(
                        n_rows[buf, r0 + k, pl.ds(on + L, L)], jnp.bfloat16)
                    a0, a1 = plsc.unpack(ua * na + ub * nb,
                                         format=plsc.PackFormat.INTERLEAVED)
                    acc_scr[pl.ds((bl * K + k) * L, L)] = a0 + a1
            for m in range(QB * K // L):
                nd_c[c & 1, pl.ds(q * QB * K + m * L, L)] = tree_reduce(
                    acc_scr, m * L * L)
            return 0

        lax.fori_loop(0, C // QB, q_body, 0)
        pltpu.async_copy(nd_c.at[c & 1], nd_hbm.at[w].at[c], wsem)

        for m in range(C // L):
            pd[c, pl.ds(m * L, L)] = tree_reduce(pos_scr, m * L * L)

    issue(0, 0, sems[0])

    def pair_body(h, _):
        for bf in range(2):
            c = h * 2 + bf

            @pl.when(c + 1 < NCHUNK)
            def _prefetch():
                issue(c + 1, 1 - bf, sems[1 - bf])

            drain(bf, sems[bf])

            @pl.when(c >= 2)
            def _wb_drain():
                pltpu.make_async_copy(nd_hbm.at[0, 0], nd_c.at[0], wsem).wait()

            compute(c, bf)
        return 0

    lax.fori_loop(0, NCHUNK // 2, pair_body, 0)
    pltpu.make_async_copy(nd_hbm.at[0, 0], nd_c.at[0], wsem).wait()
    pltpu.make_async_copy(nd_hbm.at[0, 0], nd_c.at[0], wsem).wait()
    pltpu.sync_copy(pd, pd_hbm.at[w])


def _tail_body(pd_ref, nd_ref, o_ref):
    pos = pd_ref[...]
    neg = nd_ref[...]
    s_pos = jnp.sum(jax.nn.log_sigmoid(pos))
    s_neg = jnp.sum(jax.nn.log_sigmoid(-neg))
    o_ref[0, 0] = -(s_pos + s_neg)


def _split_idx(v):
    return v & (SEG - 1), (v >> SEG_SHIFT) << 5


def kernel(pos_u, pos_v, neg_v, u_table, v_table):
    pu = pos_u.astype(jnp.int32)
    pv = pos_v.astype(jnp.int32)
    ng = neg_v.astype(jnp.int32).reshape(-1)

    pur_a, puo_a = _split_idx(pu)
    pvr_a, pvo_a = _split_idx(pv)
    ngr_a, ngo_a = _split_idx(ng)
    pur_a = pur_a.reshape(NW, NCHUNK, C)
    pvr_a = pvr_a.reshape(NW, NCHUNK, C)
    ngr_a = ngr_a.reshape(NW, NCHUNK * NG, G)
    puo_a = jnp.pad(puo_a.reshape(NW, NCHUNK, C), ((0, 0), (0, 0), (0, L)))
    pvo_a = jnp.pad(pvo_a.reshape(NW, NCHUNK, C), ((0, 0), (0, 0), (0, L)))
    ngo_a = jnp.pad(ngo_a.reshape(NW, NCHUNK, NEG_PER_CHUNK),
                    ((0, 0), (0, 0), (0, 2 * L)))

    ut_p, vt_p = _detranspose(u_table, v_table)

    mesh = plsc.VectorSubcoreMesh(core_axis_name="c", subcore_axis_name="s")
    pd, nd = pl.kernel(
        _sc_dots,
        out_type=(
            jax.ShapeDtypeStruct((NW, NCHUNK, C), jnp.float32),
            jax.ShapeDtypeStruct((NW, NCHUNK, NEG_PER_CHUNK), jnp.float32),
        ),
        mesh=mesh,
        compiler_params=pltpu.CompilerParams(
            needs_layout_passes=False, use_tc_tiling_on_sc=False),
        scratch_types=[
            pltpu.VMEM((NCHUNK, C), jnp.int32),
            pltpu.VMEM((NCHUNK, C + L), jnp.int32),
            pltpu.VMEM((NCHUNK, C), jnp.int32),
            pltpu.VMEM((NCHUNK, C + L), jnp.int32),
            pltpu.VMEM((NCHUNK * NG, G), jnp.int32),
            pltpu.VMEM((NCHUNK, NEG_PER_CHUNK + 2 * L), jnp.int32),
            pltpu.VMEM((2, C, DP), jnp.float32),
            pltpu.VMEM((2, C, DP), jnp.float32),
            pltpu.VMEM((2, NEG_PER_CHUNK, DP), jnp.float32),
            pltpu.VMEM((QB * K * L,), jnp.float32),
            pltpu.VMEM((C * L,), jnp.float32),
            pltpu.VMEM((NCHUNK, C), jnp.float32),
            pltpu.VMEM((2, NEG_PER_CHUNK), jnp.float32),
            pltpu.SemaphoreType.DMA,
            pltpu.SemaphoreType.DMA,
            pltpu.SemaphoreType.DMA,
        ],
    )(pur_a, puo_a, pvr_a, pvo_a, ngr_a, ngo_a, ut_p, vt_p)

    out = pl.pallas_call(
        _tail_body,
        out_shape=jax.ShapeDtypeStruct((1, 1), jnp.float32),
        out_specs=pl.BlockSpec(memory_space=pltpu.SMEM),
    )(pd.reshape(B // 128, 128), nd.reshape(B * K // 128, 128))
    return out[0, 0]


# speedup vs baseline: 1.2323x; 1.2323x over previous
import functools

import jax
import jax.numpy as jnp
from jax import lax
from jax.experimental import pallas as pl
from jax.experimental.pallas import tpu as pltpu
from jax.experimental.pallas import tpu_sc as plsc

V, D, B, K = 1000000, 64, 16384, 20
DP = 128
PACK = 4
SEG = 1 << 18
SEG_SHIFT = 18
WPR = DP // PACK
TW = 2048
SEGB = SEG // TW
NC, NS = 2, 16
NW = NC * NS
BPW = B // NW
C = 16
NCHUNK = BPW // C
G = 64
NEG_PER_CHUNK = C * K
NG = NEG_PER_CHUNK // G
L = 16
QB = 4


def _detranspose_body(*refs):
    ins, outs = refs[:2 * PACK], refs[2 * PACK:]
    for t in range(2):
        zs = []
        for p in range(PACK):
            xi = lax.bitcast_convert_type(ins[t * PACK + p][...], jnp.uint32)
            zs.append((xi[0:WPR, :] >> 16)
                      | (xi[WPR:2 * WPR, :] & jnp.uint32(0xFFFF0000)))
        zc = jnp.concatenate(zs, axis=0)
        outs[t][...] = lax.bitcast_convert_type(zc.T, jnp.float32)


def _detranspose(u_table, v_table):
    max_blk = V // TW
    in_specs = []
    for _ in range(2):
        for p in range(PACK):
            in_specs.append(pl.BlockSpec(
                (D, TW),
                functools.partial(
                    lambda p_, j: (0, jnp.minimum(j + SEGB * p_, max_blk)), p)))
    return pl.pallas_call(
        _detranspose_body,
        grid=(SEGB,),
        compiler_params=pltpu.CompilerParams(
            fuse_transposed_lhs_in_matmul=True),
        in_specs=in_specs,
        out_specs=[
            pl.BlockSpec((TW, DP), lambda j: (j, 0)),
            pl.BlockSpec((TW, DP), lambda j: (j, 0)),
        ],
        out_shape=(
            jax.ShapeDtypeStruct((SEG, DP), jnp.float32),
            jax.ShapeDtypeStruct((SEG, DP), jnp.float32),
        ),
    )(*([u_table.T] * PACK + [v_table.T] * PACK))


def _unpack2(vec):
    return plsc.unpack(plsc.bitcast(vec, jnp.bfloat16),
                       format=plsc.PackFormat.INTERLEAVED)


def _sc_dots(pur_hbm, puo_hbm, pvr_hbm, pvo_hbm, ngr_hbm, ngo_hbm,
             ut_hbm, vt_hbm,
             pd_hbm, nd_hbm,
             pur, puo, pvr, pvo, ngr, ngo,
             u_rows, v_rows, n_rows, acc_scr, pos_scr, pd, nd_c,
             sem0, sem1, wsem):
    w = lax.axis_index("s") * NC + lax.axis_index("c")
    pltpu.sync_copy(pur_hbm.at[w], pur)
    pltpu.sync_copy(puo_hbm.at[w], puo)
    pltpu.sync_copy(pvr_hbm.at[w], pvr)
    pltpu.sync_copy(pvo_hbm.at[w], pvo)
    pltpu.sync_copy(ngr_hbm.at[w], ngr)
    pltpu.sync_copy(ngo_hbm.at[w], ngo)

    sems = (sem0, sem1)
    lanes = lax.iota(jnp.int32, L)

    def issue(c, buf, sem):
        pltpu.async_copy(ut_hbm.at[pur.at[c]], u_rows.at[buf], sem)
        pltpu.async_copy(vt_hbm.at[pvr.at[c]], v_rows.at[buf], sem)
        for g in range(NG):
            pltpu.async_copy(
                vt_hbm.at[ngr.at[c * NG + g]],
                n_rows.at[buf].at[pl.ds(g * G, G)], sem)

    def drain(buf, sem):
        pltpu.make_async_copy(ut_hbm.at[pl.ds(0, C)], u_rows.at[buf], sem).wait()
        pltpu.make_async_copy(ut_hbm.at[pl.ds(0, C)], v_rows.at[buf], sem).wait()
        pltpu.make_async_copy(vt_hbm.at[pl.ds(0, NEG_PER_CHUNK)],
                              n_rows.at[buf], sem).wait()

    def tree_reduce(scr, base):
        gs = [plsc.load_gather(scr, [lanes * L + (base + j)]) for j in range(L)]
        while len(gs) > 1:
            gs = [gs[i] + gs[i + 1] for i in range(0, len(gs), 2)]
        return gs[0]

    def compute(c, buf):
        def q_body(q, _):
            pov = puo[c, pl.ds(q * QB, L)]
            pvv = pvo[c, pl.ds(q * QB, L)]
            for bl in range(QB):
                b = q * QB + bl
                ob = pov[bl]
                ua = plsc.bitcast(u_rows[buf, b, pl.ds(ob, L)], jnp.bfloat16)
                ub = plsc.bitcast(u_rows[buf, b, pl.ds(ob + L, L)], jnp.bfloat16)
                ov = pvv[bl]
                va = plsc.bitcast(v_rows[buf, b, pl.ds(ov, L)], jnp.bfloat16)
                vb = plsc.bitcast(v_rows[buf, b, pl.ds(ov + L, L)], jnp.bfloat16)
                p0, p1 = plsc.unpack(ua * va + ub * vb,
                                     format=plsc.PackFormat.INTERLEAVED)
                pos_scr[pl.ds(b * L, L)] = p0 + p1
                r0 = b * K
                nov1 = ngo[c, pl.ds(r0, L)]
                nov2 = ngo[c, pl.ds(r0 + L, L)]
                for k in range(K):
                    on = nov1[k] if k < L else nov2[k - L]
                    na = plsc.bitcast(
                        n_rows[buf, r0 + k, pl.ds(on, L)], jnp.bfloat16)
                    nb = plsc.bitcast(
                        n_rows[buf, r0 + k, pl.ds(on + L, L)], jnp.bfloat16)
                    a0, a1 = plsc.unpack(ua * na + ub * nb,
                                         format=plsc.PackFormat.INTERLEAVED)
                    acc_scr[pl.ds((bl * K + k) * L, L)] = a0 + a1
            for m in range(QB * K // L):
                nd_c[c & 1, pl.ds(q * QB * K + m * L, L)] = tree_reduce(
                    acc_scr, m * L * L)
            return 0

        lax.fori_loop(0, C // QB, q_body, 0)
        pltpu.async_copy(nd_c.at[c & 1], nd_hbm.at[w].at[c], wsem)

        for m in range(C // L):
            pd[c, pl.ds(m * L, L)] = tree_reduce(pos_scr, m * L * L)

    issue(0, 0, sems[0])

    def pair_body(h, _):
        for bf in range(2):
            c = h * 2 + bf

            @pl.when(c + 1 < NCHUNK)
            def _prefetch():
                issue(c + 1, 1 - bf, sems[1 - bf])

            drain(bf, sems[bf])

            @pl.when(c >= 2)
            def _wb_drain():
                pltpu.make_async_copy(nd_hbm.at[0, 0], nd_c.at[0], wsem).wait()

            compute(c, bf)
        return 0

    lax.fori_loop(0, NCHUNK // 2, pair_body, 0)
    pltpu.make_async_copy(nd_hbm.at[0, 0], nd_c.at[0], wsem).wait()
    pltpu.make_async_copy(nd_hbm.at[0, 0], nd_c.at[0], wsem).wait()
    pltpu.sync_copy(pd, pd_hbm.at[w])


def _tail_body(pd_ref, nd_ref, o_ref):
    pos = pd_ref[...]
    neg = nd_ref[...]
    s_pos = jnp.sum(jax.nn.log_sigmoid(pos))
    s_neg = jnp.sum(jax.nn.log_sigmoid(-neg))
    o_ref[0, 0] = -(s_pos + s_neg)


def _split_idx(v):
    return v & (SEG - 1), (v >> SEG_SHIFT) << 5


def kernel(pos_u, pos_v, neg_v, u_table, v_table):
    pu = pos_u.astype(jnp.int32)
    pv = pos_v.astype(jnp.int32)
    ng = neg_v.astype(jnp.int32).reshape(-1)

    pur_a, puo_a = _split_idx(pu)
    pvr_a, pvo_a = _split_idx(pv)
    ngr_a, ngo_a = _split_idx(ng)
    pur_a = pur_a.reshape(NW, NCHUNK, C)
    pvr_a = pvr_a.reshape(NW, NCHUNK, C)
    ngr_a = ngr_a.reshape(NW, NCHUNK * NG, G)
    puo_a = jnp.pad(puo_a.reshape(NW, NCHUNK, C), ((0, 0), (0, 0), (0, L)))
    pvo_a = jnp.pad(pvo_a.reshape(NW, NCHUNK, C), ((0, 0), (0, 0), (0, L)))
    ngo_a = jnp.pad(ngo_a.reshape(NW, NCHUNK, NEG_PER_CHUNK),
                    ((0, 0), (0, 0), (0, 2 * L)))

    ut_p, vt_p = _detranspose(u_table, v_table)

    mesh = plsc.VectorSubcoreMesh(core_axis_name="c", subcore_axis_name="s")
    pd, nd = pl.kernel(
        _sc_dots,
        out_type=(
            jax.ShapeDtypeStruct((NW, NCHUNK, C), jnp.float32),
            jax.ShapeDtypeStruct((NW, NCHUNK, NEG_PER_CHUNK), jnp.float32),
        ),
        mesh=mesh,
        compiler_params=pltpu.CompilerParams(
            needs_layout_passes=False, use_tc_tiling_on_sc=False),
        scratch_types=[
            pltpu.VMEM((NCHUNK, C), jnp.int32),
            pltpu.VMEM((NCHUNK, C + L), jnp.int32),
            pltpu.VMEM((NCHUNK, C), jnp.int32),
            pltpu.VMEM((NCHUNK, C + L), jnp.int32),
            pltpu.VMEM((NCHUNK * NG, G), jnp.int32),
            pltpu.VMEM((NCHUNK, NEG_PER_CHUNK + 2 * L), jnp.int32),
            pltpu.VMEM((2, C, DP), jnp.float32),
            pltpu.VMEM((2, C, DP), jnp.float32),
            pltpu.VMEM((2, NEG_PER_CHUNK, DP), jnp.float32),
            pltpu.VMEM((QB * K * L,), jnp.float32),
            pltpu.VMEM((C * L,), jnp.float32),
            pltpu.VMEM((NCHUNK, C), jnp.float32),
            pltpu.VMEM((2, NEG_PER_CHUNK), jnp.float32),
            pltpu.SemaphoreType.DMA,
            pltpu.SemaphoreType.DMA,
            pltpu.SemaphoreType.DMA,
        ],
    )(pur_a, puo_a, pvr_a, pvo_a, ngr_a, ngo_a, ut_p, vt_p)

    out = pl.pallas_call(
        _tail_body,
        out_shape=jax.ShapeDtypeStruct((1, 1), jnp.float32),
        out_specs=pl.BlockSpec(memory_space=pltpu.SMEM),
    )(pd.reshape(B // 128, 128), nd.reshape(B * K // 128, 128))
    return out[0, 0]
